# Initial kernel scaffold; baseline (speedup 1.0000x reference)
import jax
import jax.numpy as jnp
from jax.experimental import pallas as pl


def kernel(x, w_g):
    raise NotImplementedError("write your pallas kernel here")


# speedup vs baseline: 2.3596x; 2.3596x over previous
import functools

import jax
import jax.numpy as jnp
from jax.experimental import pallas as pl

N_TOK = 32768
D = 768
N_EXP = 64
BLK = 2048

NEG_INF = float("-inf")


def _router_kernel(x_ref, w_ref, vals_ref, idx_ref):
    x = x_ref[...]
    w = w_ref[...]
    logits = jax.lax.dot_general(
        x, w,
        dimension_numbers=(((1,), (1,)), ((), ())),
        preferred_element_type=jnp.float32,
    )

    col = jax.lax.broadcasted_iota(jnp.int32, logits.shape, 1)
    m1 = jnp.max(logits, axis=1, keepdims=True)
    i1 = jnp.min(jnp.where(logits == m1, col, N_EXP), axis=1, keepdims=True)
    masked = jnp.where(col == i1, NEG_INF, logits)
    m2 = jnp.max(masked, axis=1, keepdims=True)
    i2 = jnp.min(jnp.where(masked == m2, col, N_EXP), axis=1, keepdims=True)

    vals_ref[...] = jnp.concatenate([m1, m2], axis=1)
    idx_ref[...] = jnp.concatenate([i1, i2], axis=1)


@functools.partial(jax.jit, static_argnames=())
def kernel(x, w_g):
    grid = (N_TOK // BLK,)
    vals, idx = pl.pallas_call(
        _router_kernel,
        grid=grid,
        in_specs=[
            pl.BlockSpec((BLK, D), lambda i: (i, 0)),
            pl.BlockSpec((N_EXP, D), lambda i: (0, 0)),
        ],
        out_specs=[
            pl.BlockSpec((BLK, 2), lambda i: (i, 0)),
            pl.BlockSpec((BLK, 2), lambda i: (i, 0)),
        ],
        out_shape=[
            jax.ShapeDtypeStruct((N_TOK, 2), jnp.float32),
            jax.ShapeDtypeStruct((N_TOK, 2), jnp.int32),
        ],
    )(x, w_g)
    return (vals, idx)
